# baseline (speedup 1.0000x reference)
import functools

import jax
import jax.numpy as jnp
import numpy as np
from jax import lax
from jax.experimental import pallas as pl
from jax.experimental.pallas import tpu as pltpu
from jax.experimental.pallas import tpu_sc as plsc

_NUM_ACT = 192
_SIZE = 256
_RANGE = 4.0
_GRID = 2.0 * _RANGE / (_SIZE - 1)

_GRID_F = np.float32(_GRID)
_HALF_GRID = np.float32(_GRID_F / np.float32(2.0))
_INV_GRID = np.float32(np.float32(1.0) / _GRID_F)
_LO = np.float32(-(_GRID * (_SIZE // 2)))
_HI = np.float32(_GRID * (_SIZE // 2 - 2))
_T_OFF = np.float32(127.5)
_TBMAX = np.nextafter(np.float32(254.0), np.float32(0.0), dtype=np.float32)

_ROW = 224 * 224
_NROWS = 2 * _NUM_ACT
_NWORKERS = 32
_RPW = _NROWS // _NWORKERS
_CPR = 4
_CHUNK = _ROW // _CPR
_CPW = _RPW * _CPR
_TABLE = _NUM_ACT * _SIZE


def _clip_body(cs_ref, out_ref):
    cs = cs_ref[...]
    shifted = jnp.concatenate([cs[:, 1:], cs[:, _SIZE - 1:]], axis=1)
    d = jnp.maximum(shifted - cs, 0.0)
    col = lax.broadcasted_iota(jnp.int32, (_NUM_ACT, _SIZE), 1)
    d = jnp.where((col == 0) | (col == _SIZE - 2), 0.0, d)
    r = lax.broadcasted_iota(jnp.int32, (_SIZE, _SIZE), 0)
    c = lax.broadcasted_iota(jnp.int32, (_SIZE, _SIZE), 1)
    tri = (r < c).astype(jnp.float32)
    new_cs = lax.dot_general(
        d, tri, (((1,), (0,)), ((), ())),
        precision=lax.Precision.HIGHEST,
        preferred_element_type=jnp.float32,
    )
    out_ref[...] = new_cs - new_cs[:, _SIZE // 2:_SIZE // 2 + 1] + _HALF_GRID


_clip_table = pl.pallas_call(
    _clip_body,
    out_shape=jax.ShapeDtypeStruct((_NUM_ACT, _SIZE), jnp.float32),
)


_BH = 112
_BW = 128
_CPP = (224 // _BH) * 2
_NCHUNK = _NROWS * _CPP
_CPW2 = _NCHUNK // _NWORKERS


def _sc_body(table_hbm, x_hbm, out_hbm, table_v,
             xb0, xb1, ob0, ob1, tsem, is0, is1, os0, os1):
    wid = lax.axis_index("s") * 2 + lax.axis_index("c")
    kb = wid * _CPW2

    _W1 = 224 - _BW

    def src(k, par):
        p = k // _CPP
        q = lax.rem(k, _CPP)
        h0 = (q // 2) * _BH
        w0 = lax.rem(q, 2) * _BW
        return x_hbm.at[p, pl.ds(h0, _BH), pl.ds(w0, _BW)]

    def dst(k, par):
        p = k // _CPP
        q = lax.rem(k, _CPP)
        h0 = (q // 2) * _BH
        w0 = lax.rem(q, 2) * _BW
        return out_hbm.at[p, pl.ds(h0, _BH), pl.ds(w0, _BW)]

    tdesc = pltpu.async_copy(table_hbm, table_v, tsem)
    pltpu.async_copy(src(kb, 0), xb0, is0)
    pltpu.async_copy(src(kb + 1, 1), xb1, is1)
    tdesc.wait()

    bufs = (
        (xb0, ob0, is0, os0, 0, _BW // 16),
        (xb1, ob1, is1, os1, 1, _W1 // 16),
    )
    nsteps = _CPW2 // 2

    def compute_block(xb, ob, base2, nvec):
        t0 = table_v.at[pl.ds(base2, _SIZE)]

        @plsc.parallel_loop(0, _BH, 1, unroll=6)
        def rowloop(r):
            for j in range(nvec):
                c0 = j * 16
                tb0 = xb[r, pl.ds(c0, 16)] * _INV_GRID + _T_OFF
                tb = jnp.minimum(jnp.maximum(tb0, 0.0), _TBMAX)
                iy = tb.astype(jnp.int32)
                f = tb0 - iy.astype(jnp.float32)
                g0 = plsc.load_gather(t0, [iy])
                g1 = plsc.load_gather(t0, [iy + 1])
                ob[r, pl.ds(c0, 16)] = g0 + f * (g1 - g0)

    def step(s, carry):
        for (xb, ob, isem, osem, par, nvec) in bufs:
            k = kb + s * 2 + par
            base2 = lax.rem(k // _CPP, _NUM_ACT) * _SIZE
            pltpu.make_async_copy(src(kb, par), xb, isem).wait()

            @pl.when(s > 0)
            def _():
                pltpu.make_async_copy(ob, dst(kb, par), osem).wait()

            compute_block(xb, ob, base2, nvec)

            pltpu.async_copy(ob, dst(k, par), osem)

            @pl.when(s < nsteps - 1)
            def _():
                pltpu.async_copy(src(k + 2, par), xb, isem)

        return carry

    lax.fori_loop(0, nsteps, step, 0)
    pltpu.make_async_copy(ob0, dst(kb, 0), os0).wait()
    pltpu.make_async_copy(ob1, dst(kb + 1, 1), os1).wait()


@functools.cache
def _make_spline_sc():
    return pl.kernel(
        _sc_body,
        out_type=jax.ShapeDtypeStruct((_NROWS, 224, 224), jnp.float32),
        mesh=plsc.VectorSubcoreMesh(core_axis_name="c", subcore_axis_name="s"),
        compiler_params=pltpu.CompilerParams(
            needs_layout_passes=False, use_tc_tiling_on_sc=True),
        scratch_types=[
            pltpu.VMEM((_TABLE,), jnp.float32),
            pltpu.VMEM((_BH, _BW), jnp.float32),
            pltpu.VMEM((_BH, _BW), jnp.float32),
            pltpu.VMEM((_BH, _BW), jnp.float32),
            pltpu.VMEM((_BH, _BW), jnp.float32),
            pltpu.SemaphoreType.DMA,
            pltpu.SemaphoreType.DMA,
            pltpu.SemaphoreType.DMA,
            pltpu.SemaphoreType.DMA,
            pltpu.SemaphoreType.DMA,
        ],
    )


@jax.jit
def kernel(x, coefficients_vect):
    cs = coefficients_vect.reshape(_NUM_ACT, _SIZE)
    table = _clip_table(cs)
    out = _make_spline_sc()(table.reshape(-1), x.reshape(_NROWS, 224, 224))
    return out.reshape(x.shape)


# speedup vs baseline: 1.1301x; 1.1301x over previous
import functools

import jax
import jax.numpy as jnp
import numpy as np
from jax import lax
from jax.experimental import pallas as pl
from jax.experimental.pallas import tpu as pltpu
from jax.experimental.pallas import tpu_sc as plsc

_NUM_ACT = 192
_SIZE = 256
_RANGE = 4.0
_GRID = 2.0 * _RANGE / (_SIZE - 1)

_GRID_F = np.float32(_GRID)
_HALF_GRID = np.float32(_GRID_F / np.float32(2.0))
_INV_GRID = np.float32(np.float32(1.0) / _GRID_F)
_LO = np.float32(-(_GRID * (_SIZE // 2)))
_HI = np.float32(_GRID * (_SIZE // 2 - 2))
_T_OFF = np.float32(127.5)
_TBMAX = np.nextafter(np.float32(254.0), np.float32(0.0), dtype=np.float32)

_ROW = 224 * 224
_NROWS = 2 * _NUM_ACT
_NWORKERS = 32
_RPW = _NROWS // _NWORKERS
_CPR = 4
_CHUNK = _ROW // _CPR
_CPW = _RPW * _CPR
_TABLE = _NUM_ACT * _SIZE


def _clip_body(cs_ref, out_ref):
    cs = cs_ref[...]
    shifted = jnp.concatenate([cs[:, 1:], cs[:, _SIZE - 1:]], axis=1)
    d = jnp.maximum(shifted - cs, 0.0)
    col = lax.broadcasted_iota(jnp.int32, (_NUM_ACT, _SIZE), 1)
    d = jnp.where((col == 0) | (col == _SIZE - 2), 0.0, d)
    r = lax.broadcasted_iota(jnp.int32, (_SIZE, _SIZE), 0)
    c = lax.broadcasted_iota(jnp.int32, (_SIZE, _SIZE), 1)
    tri = (r < c).astype(jnp.float32)
    new_cs = lax.dot_general(
        d, tri, (((1,), (0,)), ((), ())),
        precision=lax.Precision.HIGHEST,
        preferred_element_type=jnp.float32,
    )
    out_ref[...] = new_cs - new_cs[:, _SIZE // 2:_SIZE // 2 + 1] + _HALF_GRID


_clip_table = pl.pallas_call(
    _clip_body,
    out_shape=jax.ShapeDtypeStruct((_NUM_ACT, _SIZE), jnp.float32),
)


_BH = 112
_BW = 128
_CPP = (224 // _BH) * 2
_NCHUNK = _NROWS * _CPP
_CPW2 = _NCHUNK // _NWORKERS


def _sc_body(table_hbm, x_hbm, out_hbm, table_v,
             xb0, xb1, ob0, ob1, tsem, is0, is1, os0, os1):
    wid = lax.axis_index("s") * 2 + lax.axis_index("c")
    kb = wid * _CPW2

    _W1 = 224 - _BW

    def src(k, par):
        p = k // _CPP
        q = lax.rem(k, _CPP)
        h0 = (q // 2) * _BH
        w0 = lax.rem(q, 2) * _BW
        return x_hbm.at[p, pl.ds(h0, _BH), pl.ds(w0, _BW)]

    def dst(k, par):
        p = k // _CPP
        q = lax.rem(k, _CPP)
        h0 = (q // 2) * _BH
        w0 = lax.rem(q, 2) * _BW
        return out_hbm.at[p, pl.ds(h0, _BH), pl.ds(w0, _BW)]

    tdesc = pltpu.async_copy(table_hbm, table_v, tsem)
    pltpu.async_copy(src(kb, 0), xb0, is0)
    pltpu.async_copy(src(kb + 1, 1), xb1, is1)
    tdesc.wait()

    bufs = (
        (xb0, ob0, is0, os0, 0, _BW // 16),
        (xb1, ob1, is1, os1, 1, _W1 // 16),
    )
    nsteps = _CPW2 // 2

    def compute_block(xb, ob, base2, nvec):
        t0 = table_v.at[pl.ds(base2, _SIZE)]

        @plsc.parallel_loop(0, _BH, 1, unroll=4)
        def rowloop(r):
            for j in range(nvec):
                c0 = j * 16
                tb0 = xb[r, pl.ds(c0, 16)] * _INV_GRID + _T_OFF
                tb = jnp.minimum(jnp.maximum(tb0, 0.0), _TBMAX)
                iy = tb.astype(jnp.int32)
                f = tb0 - iy.astype(jnp.float32)
                g0 = plsc.load_gather(t0, [iy])
                g1 = plsc.load_gather(t0, [iy + 1])
                ob[r, pl.ds(c0, 16)] = g0 + f * (g1 - g0)

    def step(s, carry):
        for (xb, ob, isem, osem, par, nvec) in bufs:
            k = kb + s * 2 + par
            base2 = lax.rem(k // _CPP, _NUM_ACT) * _SIZE
            pltpu.make_async_copy(src(kb, par), xb, isem).wait()

            @pl.when(s > 0)
            def _():
                pltpu.make_async_copy(ob, dst(kb, par), osem).wait()

            compute_block(xb, ob, base2, nvec)

            pltpu.async_copy(ob, dst(k, par), osem)

            @pl.when(s < nsteps - 1)
            def _():
                pltpu.async_copy(src(k + 2, par), xb, isem)

        return carry

    lax.fori_loop(0, nsteps, step, 0)
    pltpu.make_async_copy(ob0, dst(kb, 0), os0).wait()
    pltpu.make_async_copy(ob1, dst(kb + 1, 1), os1).wait()


@functools.cache
def _make_spline_sc():
    return pl.kernel(
        _sc_body,
        out_type=jax.ShapeDtypeStruct((_NROWS, 224, 224), jnp.float32),
        mesh=plsc.VectorSubcoreMesh(core_axis_name="c", subcore_axis_name="s"),
        compiler_params=pltpu.CompilerParams(
            needs_layout_passes=False, use_tc_tiling_on_sc=True),
        scratch_types=[
            pltpu.VMEM((_TABLE,), jnp.float32),
            pltpu.VMEM((_BH, _BW), jnp.float32),
            pltpu.VMEM((_BH, _BW), jnp.float32),
            pltpu.VMEM((_BH, _BW), jnp.float32),
            pltpu.VMEM((_BH, _BW), jnp.float32),
            pltpu.SemaphoreType.DMA,
            pltpu.SemaphoreType.DMA,
            pltpu.SemaphoreType.DMA,
            pltpu.SemaphoreType.DMA,
            pltpu.SemaphoreType.DMA,
        ],
    )


@jax.jit
def kernel(x, coefficients_vect):
    cs = coefficients_vect.reshape(_NUM_ACT, _SIZE)
    table = _clip_table(cs)
    out = _make_spline_sc()(table.reshape(-1), x.reshape(_NROWS, 224, 224))
    return out.reshape(x.shape)


# speedup vs baseline: 1.1307x; 1.0006x over previous
import functools

import jax
import jax.numpy as jnp
import numpy as np
from jax import lax
from jax.experimental import pallas as pl
from jax.experimental.pallas import tpu as pltpu
from jax.experimental.pallas import tpu_sc as plsc

_NUM_ACT = 192
_SIZE = 256
_RANGE = 4.0
_GRID = 2.0 * _RANGE / (_SIZE - 1)

_GRID_F = np.float32(_GRID)
_HALF_GRID = np.float32(_GRID_F / np.float32(2.0))
_INV_GRID = np.float32(np.float32(1.0) / _GRID_F)
_T_OFF = np.float32(127.5)
_TBMAX = np.nextafter(np.float32(254.0), np.float32(0.0), dtype=np.float32)

_NROWS = 2 * _NUM_ACT
_NWORKERS = 32
_TABLE = _NUM_ACT * _SIZE


def _clip_body(cs_ref, out_ref):
    cs = cs_ref[...]
    shifted = jnp.concatenate([cs[:, 1:], cs[:, _SIZE - 1:]], axis=1)
    d = jnp.maximum(shifted - cs, 0.0)
    col = lax.broadcasted_iota(jnp.int32, (_NUM_ACT, _SIZE), 1)
    d = jnp.where((col == 0) | (col == _SIZE - 2), 0.0, d)
    r = lax.broadcasted_iota(jnp.int32, (_SIZE, _SIZE), 0)
    c = lax.broadcasted_iota(jnp.int32, (_SIZE, _SIZE), 1)
    tri = (r < c).astype(jnp.float32)
    new_cs = lax.dot_general(
        d, tri, (((1,), (0,)), ((), ())),
        precision=lax.Precision.HIGHEST,
        preferred_element_type=jnp.float32,
    )
    out_ref[...] = new_cs - new_cs[:, _SIZE // 2:_SIZE // 2 + 1] + _HALF_GRID


_clip_table = pl.pallas_call(
    _clip_body,
    out_shape=jax.ShapeDtypeStruct((_NUM_ACT, _SIZE), jnp.float32),
)


_BH = 112
_BW = 128
_CPP = (224 // _BH) * 2
_NCHUNK = _NROWS * _CPP
_CPW2 = _NCHUNK // _NWORKERS


def _sc_body(table_hbm, x_hbm, out_hbm, table_v,
             xb0, xb1, ob0, ob1, tsem, is0, is1, os0, os1):
    wid = lax.axis_index("s") * 2 + lax.axis_index("c")
    kb = wid * _CPW2

    _W1 = 224 - _BW

    def src(k, par):
        p = k // _CPP
        q = lax.rem(k, _CPP)
        h0 = (q // 2) * _BH
        w0 = lax.rem(q, 2) * _BW
        return x_hbm.at[p, pl.ds(h0, _BH), pl.ds(w0, _BW)]

    def dst(k, par):
        p = k // _CPP
        q = lax.rem(k, _CPP)
        h0 = (q // 2) * _BH
        w0 = lax.rem(q, 2) * _BW
        return out_hbm.at[p, pl.ds(h0, _BH), pl.ds(w0, _BW)]

    tdesc = pltpu.async_copy(table_hbm, table_v, tsem)
    pltpu.async_copy(src(kb, 0), xb0, is0)
    pltpu.async_copy(src(kb + 1, 1), xb1, is1)
    tdesc.wait()

    bufs = (
        (xb0, ob0, is0, os0, 0, _BW // 16),
        (xb1, ob1, is1, os1, 1, _W1 // 16),
    )
    nsteps = _CPW2 // 2

    def compute_block(xb, ob, base2, nvec):
        t0 = table_v.at[pl.ds(base2, _SIZE)]

        @plsc.parallel_loop(0, _BH, 1, unroll=4)
        def rowloop(r):
            for j in range(nvec):
                c0 = j * 16
                tb0 = xb[r, pl.ds(c0, 16)] * _INV_GRID + _T_OFF
                tb = jnp.minimum(jnp.maximum(tb0, 0.0), _TBMAX)
                iy = tb.astype(jnp.int32)
                f = tb0 - iy.astype(jnp.float32)
                g0 = plsc.load_gather(t0, [iy])
                g1 = plsc.load_gather(t0, [iy + 1])
                ob[r, pl.ds(c0, 16)] = g0 + f * (g1 - g0)

    def step(s, carry):
        for (xb, ob, isem, osem, par, nvec) in bufs:
            k = kb + s * 2 + par
            base2 = lax.rem(k // _CPP, _NUM_ACT) * _SIZE
            pltpu.make_async_copy(src(kb, par), xb, isem).wait()

            @pl.when(s > 0)
            def _():
                pltpu.make_async_copy(ob, dst(kb, par), osem).wait()

            compute_block(xb, ob, base2, nvec)

            pltpu.async_copy(ob, dst(k, par), osem)

            @pl.when(s < nsteps - 1)
            def _():
                pltpu.async_copy(src(k + 2, par), xb, isem)

        return carry

    lax.fori_loop(0, nsteps, step, 0)
    pltpu.make_async_copy(ob0, dst(kb, 0), os0).wait()
    pltpu.make_async_copy(ob1, dst(kb + 1, 1), os1).wait()


@functools.cache
def _make_spline_sc():
    return pl.kernel(
        _sc_body,
        out_type=jax.ShapeDtypeStruct((_NROWS, 224, 224), jnp.float32),
        mesh=plsc.VectorSubcoreMesh(core_axis_name="c", subcore_axis_name="s"),
        compiler_params=pltpu.CompilerParams(
            needs_layout_passes=False, use_tc_tiling_on_sc=True),
        scratch_types=[
            pltpu.VMEM((_TABLE,), jnp.float32),
            pltpu.VMEM((_BH, _BW), jnp.float32),
            pltpu.VMEM((_BH, _BW), jnp.float32),
            pltpu.VMEM((_BH, _BW), jnp.float32),
            pltpu.VMEM((_BH, _BW), jnp.float32),
            pltpu.SemaphoreType.DMA,
            pltpu.SemaphoreType.DMA,
            pltpu.SemaphoreType.DMA,
            pltpu.SemaphoreType.DMA,
            pltpu.SemaphoreType.DMA,
        ],
    )


@jax.jit
def kernel(x, coefficients_vect):
    cs = coefficients_vect.reshape(_NUM_ACT, _SIZE)
    table = _clip_table(cs)
    out = _make_spline_sc()(table.reshape(-1), x.reshape(_NROWS, 224, 224))
    return out.reshape(x.shape)
